# baseline (speedup 1.0000x reference)
import functools

import jax
import jax.numpy as jnp
from jax import lax
from jax.experimental import pallas as pl
from jax.experimental.pallas import tpu as pltpu
from jax.experimental.pallas import tpu_sc as plsc


@functools.lru_cache(maxsize=None)
def _make_sweep(V, D, B):
    info = plsc.get_sparse_core_info()
    NC, NS = info.num_cores, info.num_subcores
    NW = NC * NS
    assert D == NW, "one embedding dim per vector subcore"
    assert B % 128 == 0
    VA = (V // 128) * 128
    TAIL = V - VA
    CH = min(VA, 44928)
    NCH = -(-VA // CH)
    LASTC = VA - (NCH - 1) * CH
    mesh = plsc.VectorSubcoreMesh(core_axis_name="c", subcore_axis_name="s")

    @functools.partial(
        pl.kernel,
        mesh=mesh,
        out_type=jax.ShapeDtypeStruct((D, B), jnp.float32),
        scratch_types=[
            pltpu.VMEM((B,), jnp.int32),
            pltpu.VMEM((B,), jnp.float32),
            pltpu.VMEM((D, max(TAIL, 1)), jnp.float32),
            pltpu.VMEM((CH,), jnp.float32),
            pltpu.VMEM((CH,), jnp.float32),
            pltpu.SemaphoreType.DMA,
            pltpu.SemaphoreType.DMA,
        ],
        compiler_params=pltpu.CompilerParams(needs_layout_passes=False),
    )
    def sweep_kernel(table_hbm, tail_hbm, idx_hbm, out_hbm,
                     idx_v, row_v, tail_v, buf0, buf1, sem0, sem1):
        w = lax.axis_index("s") * NC + lax.axis_index("c")
        bufs, sems = [buf0, buf1], [sem0, sem1]
        copies = {}
        copies[0] = pltpu.async_copy(
            table_hbm.at[w].at[pl.ds(0, CH)], buf0, sem0)
        idx_copy = pltpu.async_copy(idx_hbm, idx_v, sem1)
        if TAIL:
            pltpu.sync_copy(tail_hbm, tail_v)
        idx_copy.wait()
        iota16 = lax.iota(jnp.int32, 16)
        wv = iota16 * 0 + w
        UNR = 16
        for k in range(NCH):
            size_k = CH if k < NCH - 1 else LASTC
            copies[k].wait()
            if k + 1 < NCH:
                nsize = CH if k + 1 < NCH - 1 else LASTC
                copies[k + 1] = pltpu.async_copy(
                    table_hbm.at[w].at[pl.ds((k + 1) * CH, nsize)],
                    bufs[(k + 1) % 2].at[pl.ds(0, nsize)],
                    sems[(k + 1) % 2],
                )
            buf = bufs[k % 2]
            lo = k * CH
            usize = jnp.uint32(size_k)

            def scan_body(j, _, buf=buf, lo=lo, usize=usize):
                for u in range(UNR):
                    pos = j * (16 * UNR) + u * 16
                    iv = idx_v[pl.ds(pos, 16)]
                    local = iv - lo
                    m = local.astype(jnp.uint32) < usize
                    val = plsc.load_gather(buf, [local], mask=m)
                    old = row_v[pl.ds(pos, 16)]
                    row_v[pl.ds(pos, 16)] = jnp.where(m, val, old)
                return ()

            lax.fori_loop(0, B // (16 * UNR), scan_body, ())

        if TAIL:
            utail = jnp.uint32(TAIL)

            def tail_body(j, _):
                for u in range(UNR):
                    pos = j * (16 * UNR) + u * 16
                    iv = idx_v[pl.ds(pos, 16)]
                    local = iv - VA
                    m = local.astype(jnp.uint32) < utail
                    val = plsc.load_gather(tail_v, [wv, local], mask=m)
                    old = row_v[pl.ds(pos, 16)]
                    row_v[pl.ds(pos, 16)] = jnp.where(m, val, old)
                return ()

            lax.fori_loop(0, B // (16 * UNR), tail_body, ())
        pltpu.sync_copy(row_v, out_hbm.at[w])

    return sweep_kernel, VA, TAIL


def kernel(target_user_weight, user_ids):
    V, D = target_user_weight.shape
    (B,) = user_ids.shape
    sweep, VA, TAIL = _make_sweep(V, D, B)
    tail = target_user_weight[VA:, :].T if TAIL else (
        jnp.zeros((D, 1), jnp.float32))
    out_t = sweep(target_user_weight.T, tail, user_ids)
    return out_t.T
